# Initial kernel scaffold; baseline (speedup 1.0000x reference)
import jax
import jax.numpy as jnp
from jax.experimental import pallas as pl


def kernel(token_ids, embed_table, proj_weight, scale):
    raise NotImplementedError("write your pallas kernel here")


# speedup vs baseline: 1.3621x; 1.3621x over previous
import functools

import jax
import jax.numpy as jnp
from jax import lax
from jax.experimental import pallas as pl
from jax.experimental.pallas import tpu as pltpu
from jax.experimental.pallas import tpu_sc as plsc

_NC = 2
_NS = 16
_NW = _NC * _NS
_CHUNK = 128


@functools.lru_cache(maxsize=None)
def _make_gather(ntok: int, d: int):
    b_per_w = ntok // _NW
    nchunk = b_per_w // _CHUNK
    mesh = plsc.VectorSubcoreMesh(core_axis_name="c", subcore_axis_name="s")

    @functools.partial(
        pl.kernel,
        out_type=jax.ShapeDtypeStruct((ntok, d), jnp.float32),
        mesh=mesh,
        scratch_types=[
            pltpu.VMEM((nchunk, _CHUNK), jnp.int32),
            pltpu.VMEM((b_per_w, d), jnp.float32),
            pltpu.SemaphoreType.DMA,
        ],
    )
    def gather_kernel(idx_hbm, table_hbm, out_hbm, idx_v, rows_v, sem):
        wid = lax.axis_index("s") * _NC + lax.axis_index("c")
        base = wid * b_per_w
        pltpu.sync_copy(idx_hbm.at[wid], idx_v)
        copies = []
        for j in range(nchunk):
            copies.append(
                pltpu.async_copy(
                    table_hbm.at[idx_v.at[j]],
                    rows_v.at[pl.ds(j * _CHUNK, _CHUNK)],
                    sem,
                )
            )
        for c in copies:
            c.wait()
        pltpu.sync_copy(rows_v, out_hbm.at[pl.ds(base, b_per_w)])

    return gather_kernel


def _proj_body(x_ref, w_ref, s_ref, o_ref):
    o_ref[...] = (
        lax.dot_general(
            x_ref[...],
            w_ref[...],
            (((1,), (1,)), ((), ())),
            preferred_element_type=jnp.float32,
        )
        * s_ref[0]
    )


@functools.lru_cache(maxsize=None)
def _make_proj(ntok: int, d: int, m: int):
    tm = 512
    grid = (ntok // tm,)
    return pl.pallas_call(
        _proj_body,
        grid=grid,
        in_specs=[
            pl.BlockSpec((tm, d), lambda i: (i, 0)),
            pl.BlockSpec((m, d), lambda i: (0, 0)),
            pl.BlockSpec(memory_space=pltpu.SMEM),
        ],
        out_specs=pl.BlockSpec((tm, m), lambda i: (i, 0)),
        out_shape=jax.ShapeDtypeStruct((ntok, m), jnp.float32),
    )


def kernel(token_ids, embed_table, proj_weight, scale):
    b, s = token_ids.shape
    ntok = b * s
    d = embed_table.shape[1]
    m = proj_weight.shape[0]
    b_per_w = ntok // _NW
    ids = token_ids.astype(jnp.int32).reshape(_NW, b_per_w // _CHUNK, _CHUNK)
    gathered = _make_gather(ntok, d)(ids, embed_table)
    out = _make_proj(ntok, d, m)(
        gathered, proj_weight, scale.astype(jnp.float32).reshape(1)
    )
    return out.reshape(b, s, m)


# speedup vs baseline: 1.4892x; 1.0933x over previous
import functools

import jax
import jax.numpy as jnp
from jax import lax
from jax.experimental import pallas as pl
from jax.experimental.pallas import tpu as pltpu
from jax.experimental.pallas import tpu_sc as plsc

_NC = 2
_NS = 16
_NW = _NC * _NS
_CHUNK = 128


@functools.lru_cache(maxsize=None)
def _make_gather(ntok: int, d: int):
    b_per_w = ntok // _NW
    nchunk = b_per_w // _CHUNK
    mesh = plsc.VectorSubcoreMesh(core_axis_name="c", subcore_axis_name="s")

    @functools.partial(
        pl.kernel,
        out_type=jax.ShapeDtypeStruct((ntok, d), jnp.float32),
        mesh=mesh,
        scratch_types=[
            pltpu.VMEM((nchunk, _CHUNK), jnp.int32),
            pltpu.VMEM((b_per_w, d), jnp.float32),
            pltpu.SemaphoreType.DMA,
        ],
    )
    def gather_kernel(idx_hbm, table_hbm, out_hbm, idx_v, rows_v, sem):
        wid = lax.axis_index("s") * _NC + lax.axis_index("c")
        base = wid * b_per_w
        pltpu.sync_copy(idx_hbm.at[wid], idx_v)
        copies = []
        for j in range(nchunk):
            copies.append(
                pltpu.async_copy(
                    table_hbm.at[idx_v.at[j]],
                    rows_v.at[pl.ds(j * _CHUNK, _CHUNK)],
                    sem,
                )
            )
        for c in copies:
            c.wait()
        pltpu.sync_copy(rows_v, out_hbm.at[pl.ds(base, b_per_w)])

    return gather_kernel


def _proj_body(x_ref, w_ref, s_ref, o_ref):
    o_ref[...] = (
        lax.dot_general(
            x_ref[...],
            w_ref[...],
            (((1,), (1,)), ((), ())),
            preferred_element_type=jnp.float32,
        )
        * s_ref[0]
    )


@functools.lru_cache(maxsize=None)
def _make_proj(ntok: int, d: int, m: int):
    tm = 1024
    grid = (ntok // tm,)
    return pl.pallas_call(
        _proj_body,
        grid=grid,
        in_specs=[
            pl.BlockSpec((tm, d), lambda i: (i, 0)),
            pl.BlockSpec((m, d), lambda i: (0, 0)),
            pl.BlockSpec(memory_space=pltpu.SMEM),
        ],
        out_specs=pl.BlockSpec((tm, m), lambda i: (i, 0)),
        out_shape=jax.ShapeDtypeStruct((ntok, m), jnp.float32),
    )


def kernel(token_ids, embed_table, proj_weight, scale):
    b, s = token_ids.shape
    ntok = b * s
    d = embed_table.shape[1]
    m = proj_weight.shape[0]
    b_per_w = ntok // _NW
    ids = token_ids.astype(jnp.int32).reshape(_NW, b_per_w // _CHUNK, _CHUNK)
    gathered = _make_gather(ntok, d)(ids, embed_table)
    out = _make_proj(ntok, d, m)(
        gathered, proj_weight, scale.astype(jnp.float32).reshape(1)
    )
    return out.reshape(b, s, m)
